# baseline (speedup 1.0000x reference)
import functools

import jax
import jax.numpy as jnp
from jax import lax
from jax.experimental import pallas as pl
from jax.experimental.pallas import tpu as pltpu
from jax.experimental.pallas import tpu_sc as plsc

BATCH = 16384
EMBED_DIM = 32
BLOCK_W = 128
LANES = 16
NUM_WORKERS = 32
B_PER_W = BATCH // NUM_WORKERS
IDX_CHUNK = 128
GROUPS = B_PER_W // LANES


def _sc_kernel(users_hbm, product_hbm, wbb_hbm, utable_hbm, ptable_hbm,
               out_hbm, idx_u, idx_p, blk, sub_u, sub_p, rows, wbb_v, out_v,
               sem):
    nc = 2
    wid = lax.axis_index("s") * nc + lax.axis_index("c")
    base = wid * B_PER_W

    pltpu.sync_copy(users_hbm.at[pl.ds(base, B_PER_W)], idx_u)
    pltpu.sync_copy(product_hbm.at[pl.ds(base, B_PER_W)], idx_p)
    pltpu.sync_copy(wbb_hbm, wbb_v)

    def split_body(g, _):
        sl = pl.ds(g * LANES, LANES)
        v = idx_u[sl]
        blk[sl] = lax.shift_right_logical(v, 2)
        sub_u[sl] = lax.shift_left(jnp.bitwise_and(v, 3), 5)
        return ()

    lax.fori_loop(0, GROUPS, split_body, (), unroll=False)

    copies = []
    for c in range(B_PER_W // IDX_CHUNK):
        sl = pl.ds(c * IDX_CHUNK, IDX_CHUNK)
        copies.append(pltpu.async_copy(
            utable_hbm.at[blk.at[sl]], rows.at[sl], sem))
    for cp in copies:
        cp.wait()

    def user_body(g, _):
        sl = pl.ds(g * LANES, LANES)
        row_idx = g * LANES + lax.iota(jnp.int32, LANES)
        sub = sub_u[sl]
        acc = wbb_v[2 * EMBED_DIM]
        for d in range(EMBED_DIM):
            acc = acc + plsc.load_gather(rows, [row_idx, sub + d]) * wbb_v[d]
        out_v[sl] = acc
        return ()

    lax.fori_loop(0, GROUPS, user_body, (), unroll=False)

    def split_body_p(g, _):
        sl = pl.ds(g * LANES, LANES)
        v = idx_p[sl]
        blk[sl] = lax.shift_right_logical(v, 2)
        sub_p[sl] = lax.shift_left(jnp.bitwise_and(v, 3), 5)
        return ()

    lax.fori_loop(0, GROUPS, split_body_p, (), unroll=False)

    copies = []
    for c in range(B_PER_W // IDX_CHUNK):
        sl = pl.ds(c * IDX_CHUNK, IDX_CHUNK)
        copies.append(pltpu.async_copy(
            ptable_hbm.at[blk.at[sl]], rows.at[sl], sem))
    for cp in copies:
        cp.wait()

    def product_body(g, _):
        sl = pl.ds(g * LANES, LANES)
        row_idx = g * LANES + lax.iota(jnp.int32, LANES)
        sub = sub_p[sl]
        acc = out_v[sl]
        for d in range(EMBED_DIM):
            acc = acc + plsc.load_gather(rows, [row_idx, sub + d]) * wbb_v[EMBED_DIM + d]
        out_v[sl] = acc
        return ()

    lax.fori_loop(0, GROUPS, product_body, (), unroll=False)

    pltpu.sync_copy(out_v, out_hbm.at[pl.ds(base, B_PER_W)])


@jax.jit
def _run(users, product, wbb, utable4, ptable4):
    mesh = plsc.VectorSubcoreMesh(core_axis_name="c", subcore_axis_name="s")
    f = functools.partial(
        pl.kernel,
        out_type=jax.ShapeDtypeStruct((BATCH,), jnp.float32),
        mesh=mesh,
        compiler_params=pltpu.CompilerParams(
            needs_layout_passes=False, use_tc_tiling_on_sc=False),
        scratch_types=[
            pltpu.VMEM((B_PER_W,), jnp.int32),
            pltpu.VMEM((B_PER_W,), jnp.int32),
            pltpu.VMEM((B_PER_W,), jnp.int32),
            pltpu.VMEM((B_PER_W,), jnp.int32),
            pltpu.VMEM((B_PER_W,), jnp.int32),
            pltpu.VMEM((B_PER_W, BLOCK_W), jnp.float32),
            pltpu.VMEM((2 * EMBED_DIM + 1, LANES), jnp.float32),
            pltpu.VMEM((B_PER_W,), jnp.float32),
            pltpu.SemaphoreType.DMA,
        ],
    )(_sc_kernel)
    return f(users, product, wbb, utable4, ptable4)


def kernel(users, product, user_table, product_table, W, b):
    wb = jnp.concatenate([W[0], b])
    wbb = jnp.broadcast_to(wb[:, None], (2 * EMBED_DIM + 1, LANES))
    ut4 = user_table.reshape(-1, BLOCK_W)
    pt4 = product_table.reshape(-1, BLOCK_W)
    out = _run(users.astype(jnp.int32), product.astype(jnp.int32),
               wbb.astype(jnp.float32), ut4, pt4)
    return out.reshape(BATCH, 1)


# speedup vs baseline: 5.9531x; 5.9531x over previous
import functools

import jax
import jax.numpy as jnp
from jax import lax
from jax.experimental import pallas as pl
from jax.experimental.pallas import tpu as pltpu
from jax.experimental.pallas import tpu_sc as plsc

BATCH = 16384
EMBED_DIM = 32
LANES = 16
NUM_WORKERS = 32
B_PER_W = BATCH // NUM_WORKERS
IDX_CHUNK = 128
GROUPS = B_PER_W // LANES
COL_BLK = 16384


def _matvec_body(w_ref, u_ref, o_ref):
    res = lax.dot_general(w_ref[...], u_ref[...], (((1,), (0,)), ((), ())),
                          preferred_element_type=jnp.float32)
    o_ref[...] = res.reshape(-1)


def _matvec(table_t, w_row):
    n = table_t.shape[1]
    grid = (n + COL_BLK - 1) // COL_BLK
    return pl.pallas_call(
        _matvec_body,
        out_shape=jax.ShapeDtypeStruct((n,), jnp.float32),
        grid=(grid,),
        in_specs=[
            pl.BlockSpec((1, EMBED_DIM), lambda i: (0, 0)),
            pl.BlockSpec((EMBED_DIM, COL_BLK), lambda i: (0, i)),
        ],
        out_specs=pl.BlockSpec((COL_BLK,), lambda i: (i,)),
    )(w_row, table_t)


def _sc_kernel(users_hbm, product_hbm, b16_hbm, su_hbm, sp_hbm,
               out_hbm, idx_u, idx_p, suv, spv, bv, out_v, sem):
    nc = 2
    wid = lax.axis_index("s") * nc + lax.axis_index("c")
    base = wid * B_PER_W

    pltpu.sync_copy(users_hbm.at[pl.ds(base, B_PER_W)], idx_u)
    pltpu.sync_copy(product_hbm.at[pl.ds(base, B_PER_W)], idx_p)
    pltpu.sync_copy(b16_hbm, bv)

    copies = []
    for c in range(B_PER_W // IDX_CHUNK):
        sl = pl.ds(c * IDX_CHUNK, IDX_CHUNK)
        copies.append(pltpu.async_copy(
            su_hbm.at[idx_u.at[sl]], suv.at[sl], sem))
        copies.append(pltpu.async_copy(
            sp_hbm.at[idx_p.at[sl]], spv.at[sl], sem))
    for cp in copies:
        cp.wait()

    def body(g, _):
        sl = pl.ds(g * LANES, LANES)
        out_v[sl] = suv[sl] + spv[sl] + bv[...]
        return ()

    lax.fori_loop(0, GROUPS, body, (), unroll=False)

    pltpu.sync_copy(out_v, out_hbm.at[pl.ds(base, B_PER_W)])


@jax.jit
def _run(users, product, b16, user_table_t, product_table_t, wu, wp):
    su = _matvec(user_table_t, wu)
    sp = _matvec(product_table_t, wp)
    mesh = plsc.VectorSubcoreMesh(core_axis_name="c", subcore_axis_name="s")
    f = functools.partial(
        pl.kernel,
        out_type=jax.ShapeDtypeStruct((BATCH,), jnp.float32),
        mesh=mesh,
        compiler_params=pltpu.CompilerParams(
            needs_layout_passes=False, use_tc_tiling_on_sc=False),
        scratch_types=[
            pltpu.VMEM((B_PER_W,), jnp.int32),
            pltpu.VMEM((B_PER_W,), jnp.int32),
            pltpu.VMEM((B_PER_W,), jnp.float32),
            pltpu.VMEM((B_PER_W,), jnp.float32),
            pltpu.VMEM((LANES,), jnp.float32),
            pltpu.VMEM((B_PER_W,), jnp.float32),
            pltpu.SemaphoreType.DMA,
        ],
    )(_sc_kernel)
    return f(users, product, b16, su, sp)


def kernel(users, product, user_table, product_table, W, b):
    b16 = jnp.broadcast_to(b, (LANES,)).astype(jnp.float32)
    wu = W[:, :EMBED_DIM]
    wp = W[:, EMBED_DIM:]
    out = _run(users.astype(jnp.int32), product.astype(jnp.int32), b16,
               user_table.T, product_table.T, wu, wp)
    return out.reshape(BATCH, 1)


# speedup vs baseline: 8.0717x; 1.3559x over previous
import functools

import jax
import jax.numpy as jnp
from jax import lax
from jax.experimental import pallas as pl
from jax.experimental.pallas import tpu as pltpu
from jax.experimental.pallas import tpu_sc as plsc

BATCH = 16384
EMBED_DIM = 32
LANES = 16
NUM_WORKERS = 32
B_PER_W = BATCH // NUM_WORKERS
IDX_CHUNK = 128
GROUPS = B_PER_W // LANES
COL_BLK = 65536


def _matvec_body(w_ref, u_ref, o_ref):
    res = lax.dot_general(w_ref[...], u_ref[...], (((1,), (0,)), ((), ())),
                          preferred_element_type=jnp.float32)
    o_ref[...] = res.reshape(-1)


def _matvec(table_t, w_row):
    n = table_t.shape[1]
    grid = (n + COL_BLK - 1) // COL_BLK
    return pl.pallas_call(
        _matvec_body,
        out_shape=jax.ShapeDtypeStruct((n,), jnp.float32),
        grid=(grid,),
        in_specs=[
            pl.BlockSpec((1, EMBED_DIM), lambda i: (0, 0)),
            pl.BlockSpec((EMBED_DIM, COL_BLK), lambda i: (0, i)),
        ],
        out_specs=pl.BlockSpec((COL_BLK,), lambda i: (i,)),
    )(w_row, table_t)


def _sc_kernel(users_hbm, product_hbm, b16_hbm, su_hbm, sp_hbm,
               out_hbm, idx_u, idx_p, suv, spv, bv, out_v, sem):
    nc = 2
    wid = lax.axis_index("s") * nc + lax.axis_index("c")
    base = wid * B_PER_W

    pltpu.sync_copy(users_hbm.at[pl.ds(base, B_PER_W)], idx_u)
    pltpu.sync_copy(product_hbm.at[pl.ds(base, B_PER_W)], idx_p)
    pltpu.sync_copy(b16_hbm, bv)

    copies = []
    for c in range(B_PER_W // IDX_CHUNK):
        sl = pl.ds(c * IDX_CHUNK, IDX_CHUNK)
        copies.append(pltpu.async_copy(
            su_hbm.at[idx_u.at[sl]], suv.at[sl], sem))
        copies.append(pltpu.async_copy(
            sp_hbm.at[idx_p.at[sl]], spv.at[sl], sem))
    for cp in copies:
        cp.wait()

    def body(g, _):
        sl = pl.ds(g * LANES, LANES)
        out_v[sl] = suv[sl] + spv[sl] + bv[...]
        return ()

    lax.fori_loop(0, GROUPS, body, (), unroll=False)

    pltpu.sync_copy(out_v, out_hbm.at[pl.ds(base, B_PER_W)])


@jax.jit
def _run(users, product, b16, user_table_t, product_table_t, wu, wp):
    su = _matvec(user_table_t, wu)
    sp = _matvec(product_table_t, wp)
    mesh = plsc.VectorSubcoreMesh(core_axis_name="c", subcore_axis_name="s")
    f = functools.partial(
        pl.kernel,
        out_type=jax.ShapeDtypeStruct((BATCH,), jnp.float32),
        mesh=mesh,
        compiler_params=pltpu.CompilerParams(
            needs_layout_passes=False, use_tc_tiling_on_sc=False),
        scratch_types=[
            pltpu.VMEM((B_PER_W,), jnp.int32),
            pltpu.VMEM((B_PER_W,), jnp.int32),
            pltpu.VMEM((B_PER_W,), jnp.float32),
            pltpu.VMEM((B_PER_W,), jnp.float32),
            pltpu.VMEM((LANES,), jnp.float32),
            pltpu.VMEM((B_PER_W,), jnp.float32),
            pltpu.SemaphoreType.DMA,
        ],
    )(_sc_kernel)
    return f(users, product, b16, su, sp)


def kernel(users, product, user_table, product_table, W, b):
    b16 = jnp.broadcast_to(b, (LANES,)).astype(jnp.float32)
    wu = W[:, :EMBED_DIM]
    wp = W[:, EMBED_DIM:]
    out = _run(users.astype(jnp.int32), product.astype(jnp.int32), b16,
               user_table.T, product_table.T, wu, wp)
    return out.reshape(BATCH, 1)
